# Initial kernel scaffold; baseline (speedup 1.0000x reference)
import jax
import jax.numpy as jnp
from jax.experimental import pallas as pl


def kernel(x, token_table, pos_table):
    raise NotImplementedError("write your pallas kernel here")


# speedup vs baseline: 2.5292x; 2.5292x over previous
import functools

import jax
import jax.numpy as jnp
from jax import lax
from jax.experimental import pallas as pl
from jax.experimental.pallas import tpu as pltpu
from jax.experimental.pallas import tpu_sc as plsc

B = 4096
T = 200
D = 64
N = B * T

NC, NS = 2, 16
NW = NC * NS
RW = N // NW
CH = 128
NCH = RW // CH
NB = 4
DG = D // 16

_mesh = plsc.VectorSubcoreMesh(core_axis_name="c", subcore_axis_name="s")


@functools.partial(
    pl.kernel,
    out_type=jax.ShapeDtypeStruct((N, D), jnp.float32),
    mesh=_mesh,
    compiler_params=pltpu.CompilerParams(use_tc_tiling_on_sc=False),
    scratch_types=[
        pltpu.VMEM((NCH, CH), jnp.int32),
        pltpu.VMEM((T, D), jnp.float32),
        pltpu.VMEM((NB, CH, D), jnp.float32),
        pltpu.SemaphoreType.DMA((NB,)),
        pltpu.SemaphoreType.DMA((NB,)),
    ],
)
def _emb(x_hbm, tok_hbm, pos_hbm, out_hbm, idx_v, pos_v, rows_v, gsem, osem):
    w = lax.axis_index("s") * NC + lax.axis_index("c")
    row0 = w * RW

    pltpu.sync_copy(x_hbm.at[pl.ds(w * NCH, NCH)], idx_v)
    pltpu.sync_copy(pos_hbm, pos_v)

    def start_gather(c, b):
        pltpu.async_copy(tok_hbm.at[idx_v.at[c]], rows_v.at[b], gsem.at[b])

    def wait_gather(c, b):
        pltpu.make_async_copy(tok_hbm.at[idx_v.at[c]], rows_v.at[b],
                              gsem.at[b]).wait()

    def start_out(c, b):
        pltpu.async_copy(rows_v.at[b], out_hbm.at[pl.ds(row0 + c * CH, CH)],
                         osem.at[b])

    def wait_out(c, b):
        pltpu.make_async_copy(rows_v.at[b],
                              out_hbm.at[pl.ds(row0 + c * CH, CH)],
                              osem.at[b]).wait()

    def add_pos(c, b):
        base_p = lax.rem(c * CH, T)
        split = jnp.minimum(T - base_p, CH)
        rb = rows_v.at[b]

        def body_lo(r, _):
            p = base_p + r
            for d in range(DG):
                sl = pl.ds(d * 16, 16)
                rb[r, sl] = rb[r, sl] + pos_v[p, sl]
            return 0

        def body_hi(r, _):
            p = base_p + r - T
            for d in range(DG):
                sl = pl.ds(d * 16, 16)
                rb[r, sl] = rb[r, sl] + pos_v[p, sl]
            return 0

        lax.fori_loop(0, split, body_lo, 0)
        lax.fori_loop(split, CH, body_hi, 0)

    for b in range(NB):
        start_gather(b, b)

    def outer(g, _):
        for b in range(NB):
            c = g * NB + b
            wait_gather(c, b)
            add_pos(c, b)
            start_out(c, b)
        for b in range(NB):
            c = g * NB + b + NB

            @pl.when(c < NCH)
            def _():
                wait_out(c - NB, b)
                start_gather(c, b)

        return 0

    lax.fori_loop(0, NCH // NB, outer, 0)

    for b in range(NB):
        wait_out(NCH - NB + b, b)


def kernel(x, token_table, pos_table):
    x2 = x.reshape(NW * NCH, CH)
    out = _emb(x2, token_table, pos_table)
    return out.reshape(B, T, D)
